# Initial kernel scaffold; baseline (speedup 1.0000x reference)
import jax
import jax.numpy as jnp
from jax.experimental import pallas as pl


def kernel(tj, kernel):
    raise NotImplementedError("write your pallas kernel here")


# speedup vs baseline: 1.2206x; 1.2206x over previous
import functools

import jax
import jax.numpy as jnp
from jax.experimental import pallas as pl
from jax.experimental.pallas import tpu as pltpu

T_MAX = 1.0
_SC = 256


def _body(tj_row_ref, tj_col_ref, w_ref, out_ref,
          rank_ref, tnext_ref, a_ref, s_ref, m_ref, *, D, TK, TU):
    b = pl.program_id(0)
    u = pl.program_id(1)
    k = pl.program_id(2)
    nk = pl.num_programs(2)

    def col_of_b(rows):
        nb = rows.shape[1]
        lane = jax.lax.broadcasted_iota(jnp.int32, (1, nb), 1)
        return jnp.sum(jnp.where(lane == b, rows, 0.0), axis=1, keepdims=True)

    @pl.when((u == 0) & (k == 0))
    def _build():
        t_row = tj_row_ref[0]
        j_idx = jax.lax.broadcasted_iota(jnp.int32, (1, D), 1)

        def rank_step(c, acc):
            t_chunk = col_of_b(tj_col_ref[pl.ds(c * _SC, _SC), :])
            k_idx = jax.lax.broadcasted_iota(jnp.int32, (_SC, 1), 0) + c * _SC
            less = (t_chunk < t_row) | ((t_chunk == t_row) & (k_idx < j_idx))
            return acc + jnp.sum(less.astype(jnp.float32), axis=0, keepdims=True)

        rank = jax.lax.fori_loop(0, D // _SC, rank_step,
                                 jnp.zeros((1, D), jnp.float32))
        rank_ref[...] = rank

        def tnext_step(c, _):
            d_vals = jax.lax.broadcasted_iota(jnp.int32, (_SC, 1), 0) + c * _SC
            mask = rank == (d_vals + 1).astype(jnp.float32)
            s = jnp.sum(jnp.where(mask, t_row, 0.0), axis=1, keepdims=True)
            s = jnp.where(d_vals == D - 1, 1e6, s)
            tnext_ref[pl.ds(c * _SC, _SC), :] = s
            return 0

        jax.lax.fori_loop(0, D // _SC, tnext_step, 0)

        d_col = jax.lax.broadcasted_iota(jnp.int32, (D, D), 0).astype(jnp.float32)
        a_ref[...] = (rank <= d_col).astype(jnp.bfloat16)

    @pl.when(k == 0)
    def _zero():
        s_ref[...] = jnp.zeros((D, TU), jnp.float32)
        m_ref[...] = jnp.zeros((D, TU), jnp.float32)

    a_tile = a_ref[:, pl.ds(k * TK, TK)]
    w = w_ref[...]
    t_seg = col_of_b(tj_col_ref[pl.ds(k * TK, TK), :])
    wt = w * t_seg
    w_hi = w.astype(jnp.bfloat16)
    w_lo = (w - w_hi.astype(jnp.float32)).astype(jnp.bfloat16)
    wt_hi = wt.astype(jnp.bfloat16)
    wt_lo = (wt - wt_hi.astype(jnp.float32)).astype(jnp.bfloat16)

    def dot(a, bm):
        return jax.lax.dot_general(a, bm, (((1,), (0,)), ((), ())),
                                   preferred_element_type=jnp.float32)

    s_ref[...] += dot(a_tile, w_hi) + dot(a_tile, w_lo)
    m_ref[...] += dot(a_tile, wt_hi) + dot(a_tile, wt_lo)

    @pl.when(k == nk - 1)
    def _select():
        S = s_ref[...] + 1.0
        ti = m_ref[...] / S
        cond = (S > 0.0) & (ti < tnext_ref[...])
        d_iota = jax.lax.broadcasted_iota(jnp.int32, (D, TU), 0)
        first = jnp.min(jnp.where(cond, d_iota, D), axis=0, keepdims=True)
        first = jnp.where(first == D, 0, first)
        sel = d_iota == first
        val = jnp.sum(jnp.where(sel, ti, 0.0), axis=0, keepdims=True)
        out_ref[0] = jnp.where(val <= T_MAX, val, T_MAX)


def kernel(tj, kernel):
    w = kernel
    B, D = tj.shape
    U = w.shape[1]
    TK = TU = 512
    tjT = tj.T
    tj3 = tj[:, None, :]
    grid = (B, U // TU, D // TK)
    out = pl.pallas_call(
        functools.partial(_body, D=D, TK=TK, TU=TU),
        grid=grid,
        in_specs=[
            pl.BlockSpec((1, 1, D), lambda b, u, k: (b, 0, 0)),
            pl.BlockSpec((D, B), lambda b, u, k: (0, 0)),
            pl.BlockSpec((TK, TU), lambda b, u, k: (k, u)),
        ],
        out_specs=pl.BlockSpec((1, 1, TU), lambda b, u, k: (b, 0, u)),
        out_shape=jax.ShapeDtypeStruct((B, 1, U), jnp.float32),
        scratch_shapes=[
            pltpu.VMEM((1, D), jnp.float32),
            pltpu.VMEM((D, 1), jnp.float32),
            pltpu.VMEM((D, D), jnp.bfloat16),
            pltpu.VMEM((D, TU), jnp.float32),
            pltpu.VMEM((D, TU), jnp.float32),
        ],
        compiler_params=pltpu.CompilerParams(
            dimension_semantics=("arbitrary", "arbitrary", "arbitrary")),
    )(tj3, tjT, w)
    return out[:, 0, :]


# speedup vs baseline: 1.2862x; 1.0538x over previous
import functools

import jax
import jax.numpy as jnp
from jax.experimental import pallas as pl
from jax.experimental.pallas import tpu as pltpu

T_MAX = 1.0
_SC = 256


def _body(tj_row_ref, tj_col_ref, w_ref, out_ref,
          rank_ref, tnext_ref, a_ref, *, D, TU):
    b = pl.program_id(0)
    u = pl.program_id(1)

    def col_of_b(rows):
        nb = rows.shape[1]
        lane = jax.lax.broadcasted_iota(jnp.int32, (1, nb), 1)
        return jnp.sum(jnp.where(lane == b, rows, 0.0), axis=1, keepdims=True)

    @pl.when(u == 0)
    def _build():
        t_row = tj_row_ref[0]
        j_idx = jax.lax.broadcasted_iota(jnp.int32, (1, D), 1)

        def rank_step(c, acc):
            t_chunk = col_of_b(tj_col_ref[pl.ds(c * _SC, _SC), :])
            k_idx = jax.lax.broadcasted_iota(jnp.int32, (_SC, 1), 0) + c * _SC
            less = (t_chunk < t_row) | ((t_chunk == t_row) & (k_idx < j_idx))
            return acc + jnp.sum(less.astype(jnp.float32), axis=0, keepdims=True)

        rank = jax.lax.fori_loop(0, D // _SC, rank_step,
                                 jnp.zeros((1, D), jnp.float32))
        rank_ref[...] = rank

        def tnext_step(c, _):
            d_vals = jax.lax.broadcasted_iota(jnp.int32, (_SC, 1), 0) + c * _SC
            mask = rank == (d_vals + 1).astype(jnp.float32)
            s = jnp.sum(jnp.where(mask, t_row, 0.0), axis=1, keepdims=True)
            s = jnp.where(d_vals == D - 1, 1e6, s)
            tnext_ref[pl.ds(c * _SC, _SC), :] = s
            return 0

        jax.lax.fori_loop(0, D // _SC, tnext_step, 0)

        d_col = jax.lax.broadcasted_iota(jnp.int32, (D, D), 0).astype(jnp.float32)
        a_ref[...] = (rank <= d_col).astype(jnp.bfloat16)

    w = w_ref[...]
    t_col = col_of_b(tj_col_ref[...])
    wt = w * t_col
    w_hi = w.astype(jnp.bfloat16)
    w_lo = (w - w_hi.astype(jnp.float32)).astype(jnp.bfloat16)
    wt_hi = wt.astype(jnp.bfloat16)
    wt_lo = (wt - wt_hi.astype(jnp.float32)).astype(jnp.bfloat16)
    a = a_ref[...]

    def dot(x, y):
        return jax.lax.dot_general(x, y, (((1,), (0,)), ((), ())),
                                   preferred_element_type=jnp.float32)

    Sp = dot(a, w_hi) + dot(a, w_lo) + 1.0
    M = dot(a, wt_hi) + dot(a, wt_lo)

    cond = (Sp > 0.0) & (M < tnext_ref[...] * Sp)
    d_iota = jax.lax.broadcasted_iota(jnp.int32, (D, TU), 0)
    first = jnp.min(jnp.where(cond, d_iota, D), axis=0, keepdims=True)
    first = jnp.where(first == D, 0, first)
    sel = d_iota == first
    S_sel = jnp.sum(jnp.where(sel, Sp, 0.0), axis=0, keepdims=True)
    M_sel = jnp.sum(jnp.where(sel, M, 0.0), axis=0, keepdims=True)
    val = M_sel / S_sel
    out_ref[0] = jnp.where(val <= T_MAX, val, T_MAX)


def kernel(tj, kernel):
    w = kernel
    B, D = tj.shape
    U = w.shape[1]
    TU = 512
    tjT = tj.T
    tj3 = tj[:, None, :]
    grid = (B, U // TU)
    out = pl.pallas_call(
        functools.partial(_body, D=D, TU=TU),
        grid=grid,
        in_specs=[
            pl.BlockSpec((1, 1, D), lambda b, u: (b, 0, 0)),
            pl.BlockSpec((D, B), lambda b, u: (0, 0)),
            pl.BlockSpec((D, TU), lambda b, u: (0, u)),
        ],
        out_specs=pl.BlockSpec((1, 1, TU), lambda b, u: (b, 0, u)),
        out_shape=jax.ShapeDtypeStruct((B, 1, U), jnp.float32),
        scratch_shapes=[
            pltpu.VMEM((1, D), jnp.float32),
            pltpu.VMEM((D, 1), jnp.float32),
            pltpu.VMEM((D, D), jnp.bfloat16),
        ],
        compiler_params=pltpu.CompilerParams(
            dimension_semantics=("arbitrary", "arbitrary")),
    )(tj3, tjT, w)
    return out[:, 0, :]
